# baseline (speedup 1.0000x reference)
import jax
import jax.numpy as jnp
from jax import lax
from jax.experimental import pallas as pl


def _pad2d(x, p):
    C, H, W = x.shape
    zc = jnp.zeros((C, H, p), x.dtype)
    x = jnp.concatenate([zc, x, zc], axis=2)
    zr = jnp.zeros((C, p, W + 2 * p), x.dtype)
    return jnp.concatenate([zr, x, zr], axis=1)


def _spiking_layer(x, Wf, K, pad, thr):
    C, H, W = x.shape
    O = Wf.shape[0]
    HW = H * W
    C8 = -(-C // 8) * 8
    xp = _pad2d(x, pad)
    if C8 != C:
        zch = jnp.zeros((C8 - C,) + xp.shape[1:], xp.dtype)
        xp = jnp.concatenate([xp, zch], axis=0)
    ip = (xp > 0).astype(jnp.float32)
    cols = []
    for kh in range(K):
        for kw in range(K):
            s = xp[:, kh:kh + H, kw:kw + W].reshape(C8, HW)
            si = ip[:, kh:kh + H, kw:kw + W].reshape(C8, HW)
            cols.append(jnp.concatenate([si, s], axis=1))
    X = jnp.concatenate(cols, axis=0)
    P = jnp.dot(Wf, X, preferred_element_type=jnp.float32)
    pot = P[:, :HW]
    tnum = P[:, HW:]
    m = jnp.max(pot, axis=0, keepdims=True)
    cidx = lax.broadcasted_iota(jnp.int32, (O, HW), 0)
    amax = jnp.min(jnp.where(pot == m, cidx, O), axis=0, keepdims=True)
    mask = (cidx == amax) & (m > thr)
    out = jnp.where(mask, tnum / jnp.maximum(pot, 1e-6), 0.0)
    return out


def _pool2x2_flat(x, O, H, W):
    W2, H2 = W // 2, H // 2
    x = x.reshape(O * H2, 2 * W)
    y = jnp.maximum(x[:, :W], x[:, W:])
    r = lax.broadcasted_iota(jnp.int32, (W, W2), 0)
    c = lax.broadcasted_iota(jnp.int32, (W, W2), 1)
    s_even = (r == 2 * c).astype(jnp.float32)
    s_odd = (r == 2 * c + 1).astype(jnp.float32)
    z = jnp.maximum(
        jnp.dot(y, s_even, preferred_element_type=jnp.float32),
        jnp.dot(y, s_odd, preferred_element_type=jnp.float32),
    )
    return z.reshape(O, H2, W2)


def _perm_weights(wraw, C, K):
    KK = K * K
    C8 = -(-C // 8) * 8
    i = lax.broadcasted_iota(jnp.int32, (C * KK, KK * C8), 0)
    j = lax.broadcasted_iota(jnp.int32, (C * KK, KK * C8), 1)
    pm = ((i // KK == j % C8) & (i % KK == j // C8)).astype(jnp.float32)
    return jnp.dot(wraw, pm, preferred_element_type=jnp.float32)


def _csnn_kernel(spk_ref, w1_ref, w2_ref, w3_ref, out_ref):
    x = spk_ref[...]
    x = _spiking_layer(x, _perm_weights(w1_ref[...], 2, 5), 5, 2, 2.4)
    x = _pool2x2_flat(x, 30, 128, 128)
    x = _spiking_layer(x, _perm_weights(w2_ref[...], 30, 3), 3, 1, 1.0)
    x = _pool2x2_flat(x, 100, 64, 64)
    x = _spiking_layer(x, _perm_weights(w3_ref[...], 100, 3), 3, 1, 1.0)
    for i in range(32):
        out_ref[:, i, :] = x[:, i * 32:(i + 1) * 32]


def kernel(spk_in, W1, W2, W3):
    w1r = W1.reshape(30, 2 * 25)
    w2r = W2.reshape(100, 30 * 9)
    w3r = W3.reshape(200, 100 * 9)
    return pl.pallas_call(
        _csnn_kernel,
        out_shape=jax.ShapeDtypeStruct((200, 32, 32), jnp.float32),
    )(spk_in, w1r, w2r, w3r)


# speedup vs baseline: 1.4626x; 1.4626x over previous
import jax
import jax.numpy as jnp
from jax import lax
from jax.experimental import pallas as pl


def _pad2d(x, p):
    C, H, W = x.shape
    zc = jnp.zeros((C, H, p), x.dtype)
    x = jnp.concatenate([zc, x, zc], axis=2)
    zr = jnp.zeros((C, p, W + 2 * p), x.dtype)
    return jnp.concatenate([zr, x, zr], axis=1)


def _spiking_layer(x, Wf, K, pad, thr):
    C, H, W = x.shape
    O = Wf.shape[0]
    HW = H * W
    C8 = -(-C // 8) * 8
    xp = _pad2d(x, pad)
    if C8 != C:
        zch = jnp.zeros((C8 - C,) + xp.shape[1:], xp.dtype)
        xp = jnp.concatenate([xp, zch], axis=0)
    cols = [
        xp[:, kh:kh + H, kw:kw + W].reshape(C8, HW)
        for kh in range(K)
        for kw in range(K)
    ]
    Xs = jnp.concatenate(cols, axis=0)
    Xi = (Xs > 0).astype(jnp.float32)
    pot = jnp.dot(Wf, Xi, preferred_element_type=jnp.float32)
    tnum = jnp.dot(Wf, Xs, preferred_element_type=jnp.float32)
    m = jnp.max(pot, axis=0, keepdims=True)
    cidx = lax.broadcasted_iota(jnp.int32, (O, HW), 0)
    amax = jnp.min(jnp.where(pot == m, cidx, O), axis=0, keepdims=True)
    mask = (cidx == amax) & (m > thr)
    out = jnp.where(mask, tnum / jnp.maximum(pot, 1e-6), 0.0)
    return out


def _pool2x2_flat(x, O, H, W):
    W2, H2 = W // 2, H // 2
    x = x.reshape(O * H2, 2 * W)
    y = jnp.maximum(x[:, :W], x[:, W:])
    r = lax.broadcasted_iota(jnp.int32, (W, W2), 0)
    c = lax.broadcasted_iota(jnp.int32, (W, W2), 1)
    s_even = (r == 2 * c).astype(jnp.float32)
    s_odd = (r == 2 * c + 1).astype(jnp.float32)
    z = jnp.maximum(
        jnp.dot(y, s_even, preferred_element_type=jnp.float32),
        jnp.dot(y, s_odd, preferred_element_type=jnp.float32),
    )
    return z.reshape(O, H2, W2)


def _csnn_kernel(spk_ref, w1_ref, w2_ref, w3_ref, out_ref):
    x = spk_ref[...]
    x = _spiking_layer(x, w1_ref[...], 5, 2, 2.4)
    x = _pool2x2_flat(x, 30, 128, 128)
    x = _spiking_layer(x, w2_ref[...], 3, 1, 1.0)
    x = _pool2x2_flat(x, 100, 64, 64)
    x = _spiking_layer(x, w3_ref[...], 3, 1, 1.0)
    out_ref[...] = x


def _wflat(W):
    O, C, KH, KW = W.shape
    C8 = -(-C // 8) * 8
    wt = jnp.transpose(W, (0, 2, 3, 1))
    wt = jnp.pad(wt, ((0, 0), (0, 0), (0, 0), (0, C8 - C)))
    return wt.reshape(O, KH * KW * C8)


def kernel(spk_in, W1, W2, W3):
    out = pl.pallas_call(
        _csnn_kernel,
        out_shape=jax.ShapeDtypeStruct((200, 32 * 32), jnp.float32),
    )(spk_in, _wflat(W1), _wflat(W2), _wflat(W3))
    return out.reshape(200, 32, 32)


# speedup vs baseline: 1.8354x; 1.2549x over previous
import jax
import jax.numpy as jnp
from jax import lax
from jax.experimental import pallas as pl


def _pad2d(x, p):
    C, H, W = x.shape
    zc = jnp.zeros((C, H, p), x.dtype)
    x = jnp.concatenate([zc, x, zc], axis=2)
    zr = jnp.zeros((C, p, W + 2 * p), x.dtype)
    return jnp.concatenate([zr, x, zr], axis=1)


def _wta(pot, tnum, O, HW, thr):
    m = jnp.max(pot, axis=0, keepdims=True)
    cidx = lax.broadcasted_iota(jnp.int32, (O, HW), 0)
    amax = jnp.min(jnp.where(pot == m, cidx, O), axis=0, keepdims=True)
    mask = (cidx == amax) & (m > thr)
    return jnp.where(mask, tnum / jnp.maximum(pot, 1e-6), 0.0)


def _spiking_layer_w128(x, Wf, K, pad, thr):
    C, H, W = x.shape
    O = Wf.shape[0]
    HW = H * W
    C8 = -(-C // 8) * 8
    parts = [x]
    if C8 != C:
        parts.append(jnp.zeros((C8 - C, H, W), x.dtype))
    xc = jnp.concatenate(parts, axis=0) if len(parts) > 1 else x
    zv = jnp.zeros((C8, pad, W), x.dtype)
    xv = jnp.concatenate([zv, xc, zv], axis=1)
    xf = xv.reshape(C8, (H + 2 * pad) * W)
    guard = jnp.zeros((C8, 128), x.dtype)
    xf = jnp.concatenate([guard, xf, guard], axis=1)
    span = (K - 1) * W + HW
    lane = lax.broadcasted_iota(jnp.int32, (C8, span), 1) % W
    shifted = {}
    for d in range(-pad, pad + 1):
        sl = xf[:, 128 + d:128 + d + span]
        if d < 0:
            sl = jnp.where(lane >= -d, sl, 0.0)
        elif d > 0:
            sl = jnp.where(lane < W - d, sl, 0.0)
        shifted[d] = sl
    cols = [
        shifted[kw - pad][:, kh * W:kh * W + HW]
        for kh in range(K)
        for kw in range(K)
    ]
    Xs = jnp.concatenate(cols, axis=0)
    Xi = (Xs > 0).astype(jnp.float32)
    pot = jnp.dot(Wf, Xi, preferred_element_type=jnp.float32)
    tnum = jnp.dot(Wf, Xs, preferred_element_type=jnp.float32)
    return _wta(pot, tnum, O, HW, thr)


def _spiking_layer(x, Wf, K, pad, thr):
    C, H, W = x.shape
    O = Wf.shape[0]
    HW = H * W
    C8 = -(-C // 8) * 8
    xp = _pad2d(x, pad)
    if C8 != C:
        zch = jnp.zeros((C8 - C,) + xp.shape[1:], xp.dtype)
        xp = jnp.concatenate([xp, zch], axis=0)
    cols = [
        xp[:, kh:kh + H, kw:kw + W].reshape(C8, HW)
        for kh in range(K)
        for kw in range(K)
    ]
    Xs = jnp.concatenate(cols, axis=0)
    Xi = (Xs > 0).astype(jnp.float32)
    pot = jnp.dot(Wf, Xi, preferred_element_type=jnp.float32)
    tnum = jnp.dot(Wf, Xs, preferred_element_type=jnp.float32)
    return _wta(pot, tnum, O, HW, thr)


def _pool2x2_flat(x, O, H, W):
    W2, H2 = W // 2, H // 2
    x = x.reshape(O * H2, 2 * W)
    y = jnp.maximum(x[:, :W], x[:, W:])
    r = lax.broadcasted_iota(jnp.int32, (W, W2), 0)
    c = lax.broadcasted_iota(jnp.int32, (W, W2), 1)
    s_even = (r == 2 * c).astype(jnp.float32)
    s_odd = (r == 2 * c + 1).astype(jnp.float32)
    z = jnp.maximum(
        jnp.dot(y, s_even, preferred_element_type=jnp.float32),
        jnp.dot(y, s_odd, preferred_element_type=jnp.float32),
    )
    return z.reshape(O, H2, W2)


def _csnn_kernel(spk_ref, w1_ref, w2_ref, w3_ref, out_ref):
    x = spk_ref[...]
    x = _spiking_layer_w128(x, w1_ref[...], 5, 2, 2.4)
    x = _pool2x2_flat(x, 30, 128, 128)
    x = _spiking_layer(x, w2_ref[...], 3, 1, 1.0)
    x = _pool2x2_flat(x, 100, 64, 64)
    x = _spiking_layer(x, w3_ref[...], 3, 1, 1.0)
    out_ref[...] = x


def _wflat(W):
    O, C, KH, KW = W.shape
    C8 = -(-C // 8) * 8
    wt = jnp.transpose(W, (0, 2, 3, 1))
    wt = jnp.pad(wt, ((0, 0), (0, 0), (0, 0), (0, C8 - C)))
    return wt.reshape(O, KH * KW * C8)


def kernel(spk_in, W1, W2, W3):
    out = pl.pallas_call(
        _csnn_kernel,
        out_shape=jax.ShapeDtypeStruct((200, 32 * 32), jnp.float32),
    )(spk_in, _wflat(W1), _wflat(W2), _wflat(W3))
    return out.reshape(200, 32, 32)


# speedup vs baseline: 2.3617x; 1.2867x over previous
import jax
import jax.numpy as jnp
from jax import lax
from jax.experimental import pallas as pl


def _wta(pot, tnum, O, HW, thr):
    m = jnp.max(pot, axis=0, keepdims=True)
    cidx = lax.broadcasted_iota(jnp.int32, (O, HW), 0)
    amax = jnp.min(jnp.where(pot == m, cidx, O), axis=0, keepdims=True)
    mask = (cidx == amax) & (m > thr)
    return jnp.where(mask, tnum / jnp.maximum(pot, 1e-6), 0.0)


def _spiking_layer(x, Wf, K, pad, thr):
    C, H, W = x.shape
    O = Wf.shape[0]
    HW = H * W
    C8 = -(-C // 8) * 8
    parts = [x]
    if C8 != C:
        parts.append(jnp.zeros((C8 - C, H, W), x.dtype))
    xc = jnp.concatenate(parts, axis=0) if len(parts) > 1 else x
    zv = jnp.zeros((C8, pad, W), x.dtype)
    xv = jnp.concatenate([zv, xc, zv], axis=1)
    xf = xv.reshape(C8, (H + 2 * pad) * W)
    g0 = jnp.zeros((C8, 128), x.dtype)
    g1 = jnp.zeros((C8, 256), x.dtype)
    xf = jnp.concatenate([g0, xf, g1], axis=1)
    span = (K - 1) * W + 128 + HW
    lane = lax.broadcasted_iota(jnp.int32, (C8, span), 1) % W
    taps = [(kh, kw - pad) for kh in range(K) for kw in range(K)]
    shifted = {}
    for kh, d in taps:
        r = (kh * W + d) % 128
        if r not in shifted:
            sl = xf[:, r:r + span]
            if d < 0:
                sl = jnp.where(lane >= -d, sl, 0.0)
            elif d > 0:
                sl = jnp.where(lane < W - d, sl, 0.0)
            shifted[r] = sl
    cols = []
    for kh, d in taps:
        off = kh * W + d
        r = off % 128
        start = 128 + off - r
        cols.append(shifted[r][:, start:start + HW])
    Xs = jnp.concatenate(cols, axis=0)
    Xi = (Xs > 0).astype(jnp.float32)
    pot = jnp.dot(Wf, Xi, preferred_element_type=jnp.float32)
    tnum = jnp.dot(Wf, Xs, preferred_element_type=jnp.float32)
    return _wta(pot, tnum, O, HW, thr)


def _pool2x2_flat(x, O, H, W):
    W2, H2 = W // 2, H // 2
    x = x.reshape(O * H2, 2 * W)
    y = jnp.maximum(x[:, :W], x[:, W:])
    r = lax.broadcasted_iota(jnp.int32, (W, W2), 0)
    c = lax.broadcasted_iota(jnp.int32, (W, W2), 1)
    s_even = (r == 2 * c).astype(jnp.float32)
    s_odd = (r == 2 * c + 1).astype(jnp.float32)
    z = jnp.maximum(
        jnp.dot(y, s_even, preferred_element_type=jnp.float32),
        jnp.dot(y, s_odd, preferred_element_type=jnp.float32),
    )
    return z.reshape(O, H2, W2)


def _csnn_kernel(spk_ref, w1_ref, w2_ref, w3_ref, out_ref):
    x = spk_ref[...]
    x = _spiking_layer(x, w1_ref[...], 5, 2, 2.4)
    x = _pool2x2_flat(x, 30, 128, 128)
    x = _spiking_layer(x, w2_ref[...], 3, 1, 1.0)
    x = _pool2x2_flat(x, 100, 64, 64)
    x = _spiking_layer(x, w3_ref[...], 3, 1, 1.0)
    out_ref[...] = x


def _wflat(W):
    O, C, KH, KW = W.shape
    C8 = -(-C // 8) * 8
    wt = jnp.transpose(W, (0, 2, 3, 1))
    wt = jnp.pad(wt, ((0, 0), (0, 0), (0, 0), (0, C8 - C)))
    return wt.reshape(O, KH * KW * C8)


def kernel(spk_in, W1, W2, W3):
    out = pl.pallas_call(
        _csnn_kernel,
        out_shape=jax.ShapeDtypeStruct((200, 32 * 32), jnp.float32),
    )(spk_in, _wflat(W1), _wflat(W2), _wflat(W3))
    return out.reshape(200, 32, 32)
